# baseline (speedup 1.0000x reference)
import functools

import jax
import jax.numpy as jnp
from jax.experimental import pallas as pl
from jax.experimental.pallas import tpu as pltpu

_CONTRACT_RHS_DIM1 = (((1,), (1,)), ((), ()))


def _se_hwbc_kernel(x_ref, w1_ref, b1_ref, w2_ref, b2_ref, alpha_ref, o_ref,
                    *, inv_hw):
    x = x_ref[...]
    hw = x.shape[0]

    chunk = 56
    partials = [
        jnp.sum(x[s:s + chunk].astype(jnp.float32), axis=0)
        for s in range(0, hw, chunk)
    ]
    pooled = sum(partials[1:], partials[0]) * inv_hw

    h = jax.lax.dot_general(pooled, w1_ref[...], _CONTRACT_RHS_DIM1,
                            preferred_element_type=jnp.float32) + b1_ref[...]
    h = jnp.where(h > 0, h, alpha_ref[0] * h)
    g = jax.lax.dot_general(h, w2_ref[...], _CONTRACT_RHS_DIM1,
                            preferred_element_type=jnp.float32) + b2_ref[...]
    gate = jax.nn.sigmoid(g).astype(x.dtype)

    o_ref[...] = x * gate[None, :, :]


def _even_batch_tile(B, per_batch_bytes, budget_bytes):
    bt = 1
    for d in range(1, B + 1):
        if B % d == 0 and d * per_batch_bytes <= budget_bytes and B >= 2 * d:
            bt = d
    return bt


def kernel(x_nchw, w1, b1, alpha, w2, b2):
    B, C, H, W = x_nchw.shape
    HW = H * W
    Cr = w1.shape[0]
    io_dtype = x_nchw.dtype
    itemsize = jnp.dtype(io_dtype).itemsize

    x = jnp.transpose(x_nchw, (2, 3, 0, 1)).reshape(HW, B, C)

    b1_r = jnp.asarray(b1, jnp.float32).reshape(1, Cr)
    b2_r = jnp.asarray(b2, jnp.float32).reshape(1, C)
    alpha_s = jnp.asarray(alpha, jnp.float32).reshape(1)

    per_batch = HW * C * itemsize
    Bt = _even_batch_tile(B, per_batch, budget_bytes=13 * 1024 * 1024)
    grid = (B // Bt,)

    cost = pl.CostEstimate(
        flops=3 * B * C * HW + 4 * B * C * Cr,
        transcendentals=B * C,
        bytes_accessed=2 * B * C * HW * itemsize + (2 * C * Cr + C + Cr) * 4,
    )
    out = pl.pallas_call(
        functools.partial(_se_hwbc_kernel, inv_hw=1.0 / HW),
        out_shape=jax.ShapeDtypeStruct((HW, B, C), io_dtype),
        grid=grid,
        in_specs=[
            pl.BlockSpec((HW, Bt, C), lambda i: (0, i, 0)),
            pl.BlockSpec((Cr, C), lambda i: (0, 0)),
            pl.BlockSpec((1, Cr), lambda i: (0, 0)),
            pl.BlockSpec((C, Cr), lambda i: (0, 0)),
            pl.BlockSpec((1, C), lambda i: (0, 0)),
            pl.BlockSpec(memory_space=pltpu.MemorySpace.SMEM),
        ],
        out_specs=pl.BlockSpec((HW, Bt, C), lambda i: (0, i, 0)),
        compiler_params=pltpu.CompilerParams(
            dimension_semantics=("parallel",),
            vmem_limit_bytes=56 * 1024 * 1024,
        ),
        cost_estimate=cost,
    )(x, w1, b1_r, w2, b2_r, alpha_s)

    return jnp.transpose(out.reshape(H, W, B, C), (2, 3, 0, 1))
